# Initial kernel scaffold; baseline (speedup 1.0000x reference)
import jax
import jax.numpy as jnp
from jax.experimental import pallas as pl


def kernel(x, y):
    raise NotImplementedError("write your pallas kernel here")


# speedup vs baseline: 1.0167x; 1.0167x over previous
import jax
import jax.numpy as jnp
from jax import lax
from jax.experimental import pallas as pl
from jax.experimental.pallas import tpu as pltpu

B, N, M, D = 4, 4096, 4096, 3
RT = 512
T = N // RT


def _chamfer_body(x_ref, yt_ref, out_ref, colmin_ref):
    b = pl.program_id(0)
    t = pl.program_id(1)

    xb = x_ref[0]
    ybt = yt_ref[0]

    xy = lax.dot_general(
        xb, ybt, (((1,), (0,)), ((), ())),
        preferred_element_type=jnp.float32)
    x2 = jnp.sum(xb * xb, axis=1)[:, None]
    y2 = jnp.sum(ybt * ybt, axis=0)[None, :]
    d2 = jnp.maximum(x2 + y2 - 2.0 * xy, 0.0)

    scale = 1.0 / (B * N)
    rowsum = jnp.sum(jnp.min(d2, axis=1)) * scale

    @pl.when(jnp.logical_and(b == 0, t == 0))
    def _():
        out_ref[...] = jnp.zeros((1, 1), jnp.float32)

    out_ref[...] += rowsum

    colpart = jnp.min(d2, axis=0, keepdims=True)

    @pl.when(t == 0)
    def _():
        colmin_ref[...] = colpart

    @pl.when(t != 0)
    def _():
        colmin_ref[...] = jnp.minimum(colmin_ref[...], colpart)

    @pl.when(t == T - 1)
    def _():
        out_ref[...] += jnp.sum(colmin_ref[...]) * scale


def kernel(x, y):
    yt = jnp.transpose(y, (0, 2, 1))
    out = pl.pallas_call(
        _chamfer_body,
        grid=(B, T),
        in_specs=[
            pl.BlockSpec((1, RT, D), lambda b, t: (b, t, 0)),
            pl.BlockSpec((1, D, M), lambda b, t: (b, 0, 0)),
        ],
        out_specs=pl.BlockSpec((1, 1), lambda b, t: (0, 0)),
        out_shape=jax.ShapeDtypeStruct((1, 1), jnp.float32),
        scratch_shapes=[pltpu.VMEM((1, M), jnp.float32)],
        compiler_params=pltpu.CompilerParams(
            dimension_semantics=("arbitrary", "arbitrary")),
    )(x, yt)
    return out[0, 0]
